# Initial kernel scaffold; baseline (speedup 1.0000x reference)
import jax
import jax.numpy as jnp
from jax.experimental import pallas as pl


def kernel(x, edge_index, W1, b1, W2, b2):
    raise NotImplementedError("write your pallas kernel here")


# speedup vs baseline: 10.6795x; 10.6795x over previous
import functools

import jax
import jax.numpy as jnp
from jax import lax
from jax.experimental import pallas as pl
from jax.experimental.pallas import tpu as pltpu
from jax.experimental.pallas import tpu_sc as plsc

_NC = 2
_NS = 16
_NW = _NC * _NS

_K = 80
_GSZ = 8
_NG = 16
_NCHUNK = _GSZ * _NG
_EPT = _K * _NCHUNK
_BR = 2000


def _sc_mesh():
    return plsc.VectorSubcoreMesh(
        core_axis_name="c", subcore_axis_name="s",
        num_cores=_NC, num_subcores=_NS)


def _i32(v):
    return jnp.int32(v)


def _stripe(si, N):
    base = si * _i32(640)
    nst = jnp.where(base + _i32(640) <= _i32(N),
                    _i32(640 // _K), (_i32(N) - base) // _i32(_K))
    return base, nst


def _make_deg_kernel(N, Npad):
    @functools.partial(
        pl.kernel,
        out_type=jax.ShapeDtypeStruct((_NC * N,), jnp.float32),
        mesh=_sc_mesh(),
        scratch_types=[
            pltpu.VMEM((_GSZ, _K), jnp.int32),
            pltpu.VMEM((_K,), jnp.float32),
            pltpu.VMEM((_K,), jnp.float32),
            pltpu.VMEM_SHARED((Npad,), jnp.float32),
        ],
    )
    def deg_kernel(dst_hbm, out_hbm, didx, ones_v, zbuf, acc):
        ci = lax.axis_index("c")
        si = lax.axis_index("s")
        wid = ci * _i32(_NS) + si
        for j in range(_K // 16):
            ones_v[pl.ds(j * 16, 16)] = jnp.ones((16,), jnp.float32)
            zbuf[pl.ds(j * 16, 16)] = jnp.zeros((16,), jnp.float32)
        base, nst = _stripe(si, N)

        def zcopy(t, carry):
            pltpu.sync_copy(zbuf, acc.at[pl.ds(base + t * _i32(_K), _K)])
            return carry

        lax.fori_loop(_i32(0), nst, zcopy, _i32(0))
        plsc.subcore_barrier()

        def gbody(g, carry):
            off = pl.multiple_of(g * _i32(_GSZ), _GSZ)
            pltpu.sync_copy(dst_hbm.at[wid, pl.ds(off, _GSZ)], didx)

            def body(j, c2):
                pltpu.sync_copy(ones_v, acc.at[didx.at[j]], add=True)
                return c2

            return lax.fori_loop(_i32(0), _i32(_GSZ), body, carry)

        lax.fori_loop(_i32(0), _i32(_NG), gbody, _i32(0))
        plsc.subcore_barrier()

        def obody(t, carry):
            rb = base + t * _i32(_K)
            pltpu.sync_copy(acc.at[pl.ds(rb, _K)], zbuf)
            pltpu.sync_copy(zbuf, out_hbm.at[pl.ds(ci * _i32(N) + rb, _K)])
            return carry

        lax.fori_loop(_i32(0), nst, obody, _i32(0))

    return deg_kernel


def _make_row_scatter_kernel(N, Npad, D):
    @functools.partial(
        pl.kernel,
        out_type=jax.ShapeDtypeStruct((_NC, N, D), jnp.float32),
        mesh=_sc_mesh(),
        compiler_params=pltpu.CompilerParams(use_tc_tiling_on_sc=False),
        scratch_types=[
            pltpu.VMEM((_GSZ, _K), jnp.int32),
            pltpu.VMEM((_GSZ, _K), jnp.int32),
            pltpu.VMEM((_K, D), jnp.float32),
            pltpu.VMEM_SHARED((Npad, D), jnp.float32),
        ],
    )
    def scat_kernel(src_hbm, dst_hbm, g_hbm, out_hbm, sidx, didx, rows, acc):
        ci = lax.axis_index("c")
        si = lax.axis_index("s")
        wid = ci * _i32(_NS) + si
        zv = jnp.zeros((16,), jnp.float32)

        def zbody(r, carry):
            for j in range(D // 16):
                rows[r, pl.ds(j * 16, 16)] = zv
            return carry

        lax.fori_loop(_i32(0), _i32(_K), zbody, _i32(0))
        base, nst = _stripe(si, N)

        def zcopy(t, carry):
            pltpu.sync_copy(rows, acc.at[pl.ds(base + t * _i32(_K), _K)])
            return carry

        lax.fori_loop(_i32(0), nst, zcopy, _i32(0))
        plsc.subcore_barrier()

        def gbody(g, carry):
            off = pl.multiple_of(g * _i32(_GSZ), _GSZ)
            pltpu.sync_copy(src_hbm.at[wid, pl.ds(off, _GSZ)], sidx)
            pltpu.sync_copy(dst_hbm.at[wid, pl.ds(off, _GSZ)], didx)

            def body(j, c2):
                pltpu.sync_copy(g_hbm.at[sidx.at[j]], rows)
                pltpu.sync_copy(rows, acc.at[didx.at[j]], add=True)
                return c2

            return lax.fori_loop(_i32(0), _i32(_GSZ), body, carry)

        lax.fori_loop(_i32(0), _i32(_NG), gbody, _i32(0))
        plsc.subcore_barrier()

        def obody(t, carry):
            rb = base + t * _i32(_K)
            pltpu.sync_copy(acc.at[pl.ds(rb, _K)], rows)
            pltpu.sync_copy(rows, out_hbm.at[ci, pl.ds(rb, _K)])
            return carry

        lax.fori_loop(_i32(0), nst, obody, _i32(0))

    return scat_kernel


def _tc_stage1(x, W1, degT):
    N, F = x.shape
    H = W1.shape[1]
    nb = N // _BR

    def body(x_ref, w_ref, d_ref, h_ref, g_ref):
        h = jnp.dot(x_ref[...], w_ref[...], preferred_element_type=jnp.float32)
        d = d_ref[...]
        dinv = lax.rsqrt(d[:, 0:1] + d[:, 1:2] + 1.0)
        h_ref[...] = h
        g_ref[...] = h * dinv

    return pl.pallas_call(
        body,
        grid=(nb,),
        in_specs=[
            pl.BlockSpec((_BR, F), lambda i: (_i32(i), _i32(0))),
            pl.BlockSpec((F, H), lambda i: (_i32(0), _i32(0))),
            pl.BlockSpec((_BR, _NC), lambda i: (_i32(i), _i32(0))),
        ],
        out_specs=[
            pl.BlockSpec((_BR, H), lambda i: (_i32(i), _i32(0))),
            pl.BlockSpec((_BR, H), lambda i: (_i32(i), _i32(0))),
        ],
        out_shape=[jax.ShapeDtypeStruct((N, H), jnp.float32),
                   jax.ShapeDtypeStruct((N, H), jnp.float32)],
    )(x, W1, degT)


def _tc_stage2(acc1, h1, degT, b1, W2, Dp):
    N, H = h1.shape
    C2 = W2.shape[1]
    nb = N // _BR

    def body(a_ref, h_ref, d_ref, b_ref, w_ref, h2_ref, g2_ref):
        s = a_ref[0] + a_ref[1]
        d = d_ref[...]
        dinv = lax.rsqrt(d[:, 0:1] + d[:, 1:2] + 1.0)
        z = dinv * s + (dinv * dinv) * h_ref[...] + b_ref[...]
        hh = jnp.maximum(z, 0.0)
        h2 = jnp.dot(hh, w_ref[...], preferred_element_type=jnp.float32)
        h2_ref[...] = h2
        g2 = h2 * dinv
        if Dp > C2:
            g2 = jnp.concatenate(
                [g2, jnp.zeros((_BR, Dp - C2), jnp.float32)], axis=1)
        g2_ref[...] = g2

    return pl.pallas_call(
        body,
        grid=(nb,),
        in_specs=[
            pl.BlockSpec((_NC, _BR, H), lambda i: (_i32(0), _i32(i), _i32(0))),
            pl.BlockSpec((_BR, H), lambda i: (_i32(i), _i32(0))),
            pl.BlockSpec((_BR, _NC), lambda i: (_i32(i), _i32(0))),
            pl.BlockSpec((1, H), lambda i: (_i32(0), _i32(0))),
            pl.BlockSpec((H, C2), lambda i: (_i32(0), _i32(0))),
        ],
        out_specs=[
            pl.BlockSpec((_BR, C2), lambda i: (_i32(i), _i32(0))),
            pl.BlockSpec((_BR, Dp), lambda i: (_i32(i), _i32(0))),
        ],
        out_shape=[jax.ShapeDtypeStruct((N, C2), jnp.float32),
                   jax.ShapeDtypeStruct((N, Dp), jnp.float32)],
    )(acc1, h1, degT, b1, W2)


def _tc_stage3(acc2, h2, degT, b2, Dp):
    N, C2 = h2.shape
    nb = N // _BR

    def body(a_ref, h_ref, d_ref, b_ref, o_ref):
        s = (a_ref[0] + a_ref[1])[:, :C2]
        d = d_ref[...]
        dinv = lax.rsqrt(d[:, 0:1] + d[:, 1:2] + 1.0)
        z = dinv * s + (dinv * dinv) * h_ref[...] + b_ref[...]
        m = jnp.max(z, axis=1, keepdims=True)
        e = jnp.exp(z - m)
        lse = jnp.log(jnp.sum(e, axis=1, keepdims=True)) + m
        o_ref[...] = z - lse

    return pl.pallas_call(
        body,
        grid=(nb,),
        in_specs=[
            pl.BlockSpec((_NC, _BR, Dp), lambda i: (_i32(0), _i32(i), _i32(0))),
            pl.BlockSpec((_BR, C2), lambda i: (_i32(i), _i32(0))),
            pl.BlockSpec((_BR, _NC), lambda i: (_i32(i), _i32(0))),
            pl.BlockSpec((1, C2), lambda i: (_i32(0), _i32(0))),
        ],
        out_specs=pl.BlockSpec((_BR, C2), lambda i: (_i32(i), _i32(0))),
        out_shape=jax.ShapeDtypeStruct((N, C2), jnp.float32),
    )(acc2, h2, degT, b2)


def kernel(x, edge_index, W1, b1, W2, b2):
    N, F = x.shape
    E = edge_index.shape[1]
    H = W1.shape[1]
    C2 = W2.shape[1]
    Dp = 48
    assert N % _BR == 0 and N % 8 == 0
    Epad = _NW * _EPT
    assert Epad >= E
    Npad = N + _K

    ei = edge_index.astype(jnp.int32)
    pad = Epad - E
    src = jnp.concatenate([ei[0], jnp.zeros((pad,), jnp.int32)])
    dst = jnp.concatenate([ei[1], jnp.full((pad,), N, jnp.int32)])
    src = src.reshape(_NW, _NCHUNK, _K)
    dst = dst.reshape(_NW, _NCHUNK, _K)

    degp = _make_deg_kernel(N, Npad)(dst).reshape(_NC, N)
    degT = jnp.transpose(degp)

    h1, g1 = _tc_stage1(x, W1, degT)
    acc1 = _make_row_scatter_kernel(N, Npad, H)(src, dst, g1)
    h2, g2p = _tc_stage2(acc1, h1, degT, b1.reshape(1, H), W2, Dp)
    acc2 = _make_row_scatter_kernel(N, Npad, Dp)(src, dst, g2p)
    return _tc_stage3(acc2, h2, degT, b2.reshape(1, C2), Dp)


# speedup vs baseline: 13.5901x; 1.2725x over previous
import functools

import jax
import jax.numpy as jnp
from jax import lax
from jax.experimental import pallas as pl
from jax.experimental.pallas import tpu as pltpu
from jax.experimental.pallas import tpu_sc as plsc

_NC = 2
_NS = 16
_NW = _NC * _NS

_K = 80
_GSZ = 8
_RING = 2 * _GSZ
_BR = 2000


def _sc_mesh():
    return plsc.VectorSubcoreMesh(
        core_axis_name="c", subcore_axis_name="s",
        num_cores=_NC, num_subcores=_NS)


def _i32(v):
    return jnp.int32(v)


def _stripe(si, N):
    base = si * _i32(640)
    nst = jnp.where(base + _i32(640) <= _i32(N),
                    _i32(640 // _K), (_i32(N) - base) // _i32(_K))
    return base, nst


def _make_deg_kernel(N, Npad):
    @functools.partial(
        pl.kernel,
        out_type=jax.ShapeDtypeStruct((_NC * N,), jnp.float32),
        mesh=_sc_mesh(),
        scratch_types=[
            pltpu.VMEM((_GSZ, _K), jnp.int32),
            pltpu.VMEM((_K,), jnp.float32),
            pltpu.VMEM((_K,), jnp.float32),
            pltpu.VMEM_SHARED((Npad,), jnp.float32),
        ],
    )
    def deg_kernel(dst_hbm, out_hbm, didx, ones_v, zbuf, acc):
        ci = lax.axis_index("c")
        si = lax.axis_index("s")
        wid = ci * _i32(_NS) + si
        ng = dst_hbm.shape[1] // _GSZ
        for j in range(_K // 16):
            ones_v[pl.ds(j * 16, 16)] = jnp.ones((16,), jnp.float32)
            zbuf[pl.ds(j * 16, 16)] = jnp.zeros((16,), jnp.float32)
        base, nst = _stripe(si, N)

        def zcopy(t, carry):
            pltpu.sync_copy(zbuf, acc.at[pl.ds(base + t * _i32(_K), _K)])
            return carry

        lax.fori_loop(_i32(0), nst, zcopy, _i32(0))
        plsc.subcore_barrier()

        def gbody(g, carry):
            off = pl.multiple_of(g * _i32(_GSZ), _GSZ)
            pltpu.sync_copy(dst_hbm.at[wid, pl.ds(off, _GSZ)], didx)

            def body(j, c2):
                pltpu.sync_copy(ones_v, acc.at[didx.at[j]], add=True)
                return c2

            return lax.fori_loop(_i32(0), _i32(_GSZ), body, carry)

        lax.fori_loop(_i32(0), _i32(ng), gbody, _i32(0))
        plsc.subcore_barrier()

        def obody(t, carry):
            rb = base + t * _i32(_K)
            pltpu.sync_copy(acc.at[pl.ds(rb, _K)], zbuf)
            pltpu.sync_copy(zbuf, out_hbm.at[pl.ds(ci * _i32(N) + rb, _K)])
            return carry

        lax.fori_loop(_i32(0), nst, obody, _i32(0))

    return deg_kernel


def _make_row_scatter_kernel(N, Npad, D, col_split):
    @functools.partial(
        pl.kernel,
        out_type=jax.ShapeDtypeStruct((_NC, N, D), jnp.float32),
        mesh=_sc_mesh(),
        compiler_params=pltpu.CompilerParams(use_tc_tiling_on_sc=False),
        scratch_types=[
            pltpu.VMEM((_RING, _K), jnp.int32),
            pltpu.VMEM((_RING, _K), jnp.int32),
            pltpu.VMEM((2, _K, D), jnp.float32),
            pltpu.VMEM_SHARED((Npad, D), jnp.float32),
            pltpu.SemaphoreType.DMA,
            pltpu.SemaphoreType.DMA,
        ],
    )
    def scat_kernel(src_hbm, dst_hbm, g_hbm, out_hbm,
                    sidx, didx, rows, acc, gsem, isem):
        ci = lax.axis_index("c")
        si = lax.axis_index("s")
        eix = si if col_split else ci * _i32(_NS) + si
        gtab = g_hbm.at[ci] if col_split else g_hbm
        ch = src_hbm.shape[1]
        ng = ch // _GSZ
        zv = jnp.zeros((16,), jnp.float32)

        def zbody(r, carry):
            for j in range(D // 16):
                rows[_i32(0), r, pl.ds(j * 16, 16)] = zv
            return carry

        lax.fori_loop(_i32(0), _i32(_K), zbody, _i32(0))
        base, nst = _stripe(si, N)

        def zcopy(t, carry):
            pltpu.sync_copy(rows.at[_i32(0)],
                            acc.at[pl.ds(base + t * _i32(_K), _K)])
            return carry

        lax.fori_loop(_i32(0), nst, zcopy, _i32(0))
        plsc.subcore_barrier()

        pltpu.sync_copy(src_hbm.at[eix, pl.ds(0, _GSZ)],
                        sidx.at[pl.ds(0, _GSZ)])
        pltpu.sync_copy(dst_hbm.at[eix, pl.ds(0, _GSZ)],
                        didx.at[pl.ds(0, _GSZ)])
        pltpu.sync_copy(gtab.at[sidx.at[_i32(0)]], rows.at[_i32(0)])

        def step(c, carry):
            rn = lax.rem(c + _i32(1), _i32(_RING))
            pn = lax.rem(c + _i32(1), _i32(2))
            d = pltpu.async_copy(gtab.at[sidx.at[rn]], rows.at[pn], gsem)
            pltpu.sync_copy(rows.at[lax.rem(c, _i32(2))],
                            acc.at[didx.at[lax.rem(c, _i32(_RING))]],
                            add=True)
            d.wait()
            return carry

        def gbody(g, carry):
            gb = lax.rem(g, _i32(2))
            nxt = jnp.where(g + _i32(1) < _i32(ng), g + _i32(1), _i32(0))
            noff = pl.multiple_of(nxt * _i32(_GSZ), _GSZ)
            rdst = (_i32(1) - gb) * _i32(_GSZ)
            d1 = pltpu.async_copy(src_hbm.at[eix, pl.ds(noff, _GSZ)],
                                  sidx.at[pl.ds(rdst, _GSZ)], isem)
            d2 = pltpu.async_copy(dst_hbm.at[eix, pl.ds(noff, _GSZ)],
                                  didx.at[pl.ds(rdst, _GSZ)], isem)
            g8 = g * _i32(_GSZ)

            def ibody(j, c2):
                return step(g8 + j, c2)

            carry = lax.fori_loop(_i32(0), _i32(_GSZ - 1), ibody, carry)
            d1.wait()
            d2.wait()
            return step(g8 + _i32(_GSZ - 1), carry)

        lax.fori_loop(_i32(0), _i32(ng), gbody, _i32(0))
        plsc.subcore_barrier()

        def obody(t, carry):
            rb = base + t * _i32(_K)
            pltpu.sync_copy(acc.at[pl.ds(rb, _K)], rows.at[_i32(0)])
            pltpu.sync_copy(rows.at[_i32(0)], out_hbm.at[ci, pl.ds(rb, _K)])
            return carry

        lax.fori_loop(_i32(0), nst, obody, _i32(0))

    return scat_kernel


def _tc_stage1(x, W1, degT):
    N, F = x.shape
    H = W1.shape[1]
    Hh = H // _NC
    nb = N // _BR

    def body(x_ref, w_ref, d_ref, h_ref, g_ref):
        h = jnp.dot(x_ref[...], w_ref[...], preferred_element_type=jnp.float32)
        d = d_ref[...]
        dinv = lax.rsqrt(d[:, 0:1] + d[:, 1:2] + 1.0)
        h_ref[...] = h
        g = h * dinv
        g_ref[0] = g[:, :Hh]
        g_ref[1] = g[:, Hh:]

    return pl.pallas_call(
        body,
        grid=(nb,),
        in_specs=[
            pl.BlockSpec((_BR, F), lambda i: (_i32(i), _i32(0))),
            pl.BlockSpec((F, H), lambda i: (_i32(0), _i32(0))),
            pl.BlockSpec((_BR, _NC), lambda i: (_i32(i), _i32(0))),
        ],
        out_specs=[
            pl.BlockSpec((_BR, H), lambda i: (_i32(i), _i32(0))),
            pl.BlockSpec((_NC, _BR, Hh),
                         lambda i: (_i32(0), _i32(i), _i32(0))),
        ],
        out_shape=[jax.ShapeDtypeStruct((N, H), jnp.float32),
                   jax.ShapeDtypeStruct((_NC, N, Hh), jnp.float32)],
    )(x, W1, degT)


def _tc_stage2(acc1, h1, degT, b1, W2, Dp):
    N, H = h1.shape
    Hh = H // _NC
    C2 = W2.shape[1]
    nb = N // _BR

    def body(a_ref, h_ref, d_ref, b_ref, w_ref, h2_ref, g2_ref):
        s = jnp.concatenate([a_ref[0], a_ref[1]], axis=1)
        d = d_ref[...]
        dinv = lax.rsqrt(d[:, 0:1] + d[:, 1:2] + 1.0)
        z = dinv * s + (dinv * dinv) * h_ref[...] + b_ref[...]
        hh = jnp.maximum(z, 0.0)
        h2 = jnp.dot(hh, w_ref[...], preferred_element_type=jnp.float32)
        h2_ref[...] = h2
        g2 = h2 * dinv
        if Dp > C2:
            g2 = jnp.concatenate(
                [g2, jnp.zeros((_BR, Dp - C2), jnp.float32)], axis=1)
        g2_ref[...] = g2

    return pl.pallas_call(
        body,
        grid=(nb,),
        in_specs=[
            pl.BlockSpec((_NC, _BR, Hh),
                         lambda i: (_i32(0), _i32(i), _i32(0))),
            pl.BlockSpec((_BR, H), lambda i: (_i32(i), _i32(0))),
            pl.BlockSpec((_BR, _NC), lambda i: (_i32(i), _i32(0))),
            pl.BlockSpec((1, H), lambda i: (_i32(0), _i32(0))),
            pl.BlockSpec((H, C2), lambda i: (_i32(0), _i32(0))),
        ],
        out_specs=[
            pl.BlockSpec((_BR, C2), lambda i: (_i32(i), _i32(0))),
            pl.BlockSpec((_BR, Dp), lambda i: (_i32(i), _i32(0))),
        ],
        out_shape=[jax.ShapeDtypeStruct((N, C2), jnp.float32),
                   jax.ShapeDtypeStruct((N, Dp), jnp.float32)],
    )(acc1, h1, degT, b1, W2)


def _tc_stage3(acc2, h2, degT, b2, Dp):
    N, C2 = h2.shape
    nb = N // _BR

    def body(a_ref, h_ref, d_ref, b_ref, o_ref):
        s = (a_ref[0] + a_ref[1])[:, :C2]
        d = d_ref[...]
        dinv = lax.rsqrt(d[:, 0:1] + d[:, 1:2] + 1.0)
        z = dinv * s + (dinv * dinv) * h_ref[...] + b_ref[...]
        m = jnp.max(z, axis=1, keepdims=True)
        e = jnp.exp(z - m)
        lse = jnp.log(jnp.sum(e, axis=1, keepdims=True)) + m
        o_ref[...] = z - lse

    return pl.pallas_call(
        body,
        grid=(nb,),
        in_specs=[
            pl.BlockSpec((_NC, _BR, Dp),
                         lambda i: (_i32(0), _i32(i), _i32(0))),
            pl.BlockSpec((_BR, C2), lambda i: (_i32(i), _i32(0))),
            pl.BlockSpec((_BR, _NC), lambda i: (_i32(i), _i32(0))),
            pl.BlockSpec((1, C2), lambda i: (_i32(0), _i32(0))),
        ],
        out_specs=pl.BlockSpec((_BR, C2), lambda i: (_i32(i), _i32(0))),
        out_shape=jax.ShapeDtypeStruct((N, C2), jnp.float32),
    )(acc2, h2, degT, b2)


def kernel(x, edge_index, W1, b1, W2, b2):
    N, F = x.shape
    E = edge_index.shape[1]
    H = W1.shape[1]
    C2 = W2.shape[1]
    Dp = 48
    assert N % _BR == 0 and N % 8 == 0
    ept = ((E + _NW * _K * _GSZ - 1) // (_NW * _K * _GSZ)) * _K * _GSZ
    Epad = _NW * ept
    Npad = N + _K

    ei = edge_index.astype(jnp.int32)
    pad = Epad - E
    srcp = jnp.concatenate([ei[0], jnp.zeros((pad,), jnp.int32)])
    dstp = jnp.concatenate([ei[1], jnp.full((pad,), N, jnp.int32)])
    src1 = srcp.reshape(_NS, -1, _K)
    dst1 = dstp.reshape(_NS, -1, _K)
    src2 = srcp.reshape(_NW, -1, _K)
    dst2 = dstp.reshape(_NW, -1, _K)

    degp = _make_deg_kernel(N, Npad)(dst2).reshape(_NC, N)
    degT = jnp.transpose(degp)

    h1, g1 = _tc_stage1(x, W1, degT)
    acc1 = _make_row_scatter_kernel(N, Npad, H // _NC, True)(src1, dst1, g1)
    h2, g2p = _tc_stage2(acc1, h1, degT, b1.reshape(1, H), W2, Dp)
    acc2 = _make_row_scatter_kernel(N, Npad, Dp, False)(src2, dst2, g2p)
    return _tc_stage3(acc2, h2, degT, b2.reshape(1, C2), Dp)


# speedup vs baseline: 13.9953x; 1.0298x over previous
import functools

import jax
import jax.numpy as jnp
from jax import lax
from jax.experimental import pallas as pl
from jax.experimental.pallas import tpu as pltpu
from jax.experimental.pallas import tpu_sc as plsc

_NC = 2
_NS = 16
_NW = _NC * _NS

_K = 80
_GSZ = 8
_RING = 2 * _GSZ
_BR = 2000


def _sc_mesh():
    return plsc.VectorSubcoreMesh(
        core_axis_name="c", subcore_axis_name="s",
        num_cores=_NC, num_subcores=_NS)


def _i32(v):
    return jnp.int32(v)


def _stripe(si, N):
    base = si * _i32(640)
    nst = jnp.where(base + _i32(640) <= _i32(N),
                    _i32(640 // _K), (_i32(N) - base) // _i32(_K))
    return base, nst


def _make_deg_kernel(N, Npad):
    @functools.partial(
        pl.kernel,
        out_type=jax.ShapeDtypeStruct((_NC * N,), jnp.float32),
        mesh=_sc_mesh(),
        scratch_types=[
            pltpu.VMEM((_GSZ, _K), jnp.int32),
            pltpu.VMEM((_K,), jnp.float32),
            pltpu.VMEM((_K,), jnp.float32),
            pltpu.VMEM_SHARED((Npad,), jnp.float32),
        ],
    )
    def deg_kernel(dst_hbm, out_hbm, didx, ones_v, zbuf, acc):
        ci = lax.axis_index("c")
        si = lax.axis_index("s")
        wid = ci * _i32(_NS) + si
        ng = dst_hbm.shape[1] // _GSZ
        for j in range(_K // 16):
            ones_v[pl.ds(j * 16, 16)] = jnp.ones((16,), jnp.float32)
            zbuf[pl.ds(j * 16, 16)] = jnp.zeros((16,), jnp.float32)
        base, nst = _stripe(si, N)

        def zcopy(t, carry):
            pltpu.sync_copy(zbuf, acc.at[pl.ds(base + t * _i32(_K), _K)])
            return carry

        lax.fori_loop(_i32(0), nst, zcopy, _i32(0))
        plsc.subcore_barrier()

        def gbody(g, carry):
            off = pl.multiple_of(g * _i32(_GSZ), _GSZ)
            pltpu.sync_copy(dst_hbm.at[wid, pl.ds(off, _GSZ)], didx)

            def body(j, c2):
                pltpu.sync_copy(ones_v, acc.at[didx.at[j]], add=True)
                return c2

            return lax.fori_loop(_i32(0), _i32(_GSZ), body, carry)

        lax.fori_loop(_i32(0), _i32(ng), gbody, _i32(0))
        plsc.subcore_barrier()

        def obody(t, carry):
            rb = base + t * _i32(_K)
            pltpu.sync_copy(acc.at[pl.ds(rb, _K)], zbuf)
            pltpu.sync_copy(zbuf, out_hbm.at[pl.ds(ci * _i32(N) + rb, _K)])
            return carry

        lax.fori_loop(_i32(0), nst, obody, _i32(0))

    return deg_kernel


def _make_row_scatter_kernel(N, Npad, D, col_split):
    @functools.partial(
        pl.kernel,
        out_type=jax.ShapeDtypeStruct((_NC, N, D), jnp.float32),
        mesh=_sc_mesh(),
        compiler_params=pltpu.CompilerParams(use_tc_tiling_on_sc=False),
        scratch_types=[
            pltpu.VMEM((_RING, _K), jnp.int32),
            pltpu.VMEM((_RING, _K), jnp.int32),
            pltpu.VMEM((2, _K, D), jnp.float32),
            pltpu.VMEM_SHARED((Npad, D), jnp.float32),
            pltpu.SemaphoreType.DMA,
            pltpu.SemaphoreType.DMA,
            pltpu.SemaphoreType.DMA,
        ],
    )
    def scat_kernel(src_hbm, dst_hbm, g_hbm, out_hbm,
                    sidx, didx, rows, acc, gsem, isem, ssem):
        ci = lax.axis_index("c")
        si = lax.axis_index("s")
        eix = si if col_split else ci * _i32(_NS) + si
        gtab = g_hbm.at[ci] if col_split else g_hbm
        ch = src_hbm.shape[1]
        ng = ch // _GSZ
        zv = jnp.zeros((16,), jnp.float32)

        def zbody(r, carry):
            for j in range(D // 16):
                rows[_i32(0), r, pl.ds(j * 16, 16)] = zv
            return carry

        lax.fori_loop(_i32(0), _i32(_K), zbody, _i32(0))
        base, nst = _stripe(si, N)

        def zcopy(t, carry):
            pltpu.sync_copy(rows.at[_i32(0)],
                            acc.at[pl.ds(base + t * _i32(_K), _K)])
            return carry

        lax.fori_loop(_i32(0), nst, zcopy, _i32(0))
        plsc.subcore_barrier()

        pltpu.sync_copy(src_hbm.at[eix, pl.ds(0, _GSZ)],
                        sidx.at[pl.ds(0, _GSZ)])
        pltpu.sync_copy(dst_hbm.at[eix, pl.ds(0, _GSZ)],
                        didx.at[pl.ds(0, _GSZ)])
        pltpu.sync_copy(gtab.at[sidx.at[_i32(0)]], rows.at[_i32(0)])

        def step(c, carry):
            rc = lax.rem(c, _i32(_RING))
            rn = lax.rem(c + _i32(1), _i32(_RING))
            pc = lax.rem(c, _i32(2))
            pn = lax.rem(c + _i32(1), _i32(2))

            @pl.when(c > _i32(0))
            def _():
                pltpu.make_async_copy(
                    rows.at[pn], acc.at[didx.at[rc]], ssem).wait()

            d = pltpu.async_copy(gtab.at[sidx.at[rn]], rows.at[pn], gsem)
            pltpu.async_copy(rows.at[pc], acc.at[didx.at[rc]], ssem,
                             add=True)
            d.wait()
            return carry

        def gbody(g, carry):
            gb = lax.rem(g, _i32(2))
            nxt = jnp.where(g + _i32(1) < _i32(ng), g + _i32(1), _i32(0))
            noff = pl.multiple_of(nxt * _i32(_GSZ), _GSZ)
            rdst = (_i32(1) - gb) * _i32(_GSZ)
            d1 = pltpu.async_copy(src_hbm.at[eix, pl.ds(noff, _GSZ)],
                                  sidx.at[pl.ds(rdst, _GSZ)], isem)
            d2 = pltpu.async_copy(dst_hbm.at[eix, pl.ds(noff, _GSZ)],
                                  didx.at[pl.ds(rdst, _GSZ)], isem)
            g8 = g * _i32(_GSZ)

            def ibody(j, c2):
                return step(g8 + j, c2)

            carry = lax.fori_loop(_i32(0), _i32(_GSZ - 1), ibody, carry)
            d1.wait()
            d2.wait()
            return step(g8 + _i32(_GSZ - 1), carry)

        lax.fori_loop(_i32(0), _i32(ng), gbody, _i32(0))
        pltpu.make_async_copy(
            rows.at[_i32(0)], acc.at[didx.at[_i32(0)]], ssem).wait()
        plsc.subcore_barrier()

        def obody(t, carry):
            rb = base + t * _i32(_K)
            pltpu.sync_copy(acc.at[pl.ds(rb, _K)], rows.at[_i32(0)])
            pltpu.sync_copy(rows.at[_i32(0)], out_hbm.at[ci, pl.ds(rb, _K)])
            return carry

        lax.fori_loop(_i32(0), nst, obody, _i32(0))

    return scat_kernel


def _tc_stage1(x, W1, degT):
    N, F = x.shape
    H = W1.shape[1]
    Hh = H // _NC
    nb = N // _BR

    def body(x_ref, w_ref, d_ref, h_ref, g_ref):
        h = jnp.dot(x_ref[...], w_ref[...], preferred_element_type=jnp.float32)
        d = d_ref[...]
        dinv = lax.rsqrt(d[:, 0:1] + d[:, 1:2] + 1.0)
        h_ref[...] = h
        g = h * dinv
        g_ref[0] = g[:, :Hh]
        g_ref[1] = g[:, Hh:]

    return pl.pallas_call(
        body,
        grid=(nb,),
        in_specs=[
            pl.BlockSpec((_BR, F), lambda i: (_i32(i), _i32(0))),
            pl.BlockSpec((F, H), lambda i: (_i32(0), _i32(0))),
            pl.BlockSpec((_BR, _NC), lambda i: (_i32(i), _i32(0))),
        ],
        out_specs=[
            pl.BlockSpec((_BR, H), lambda i: (_i32(i), _i32(0))),
            pl.BlockSpec((_NC, _BR, Hh),
                         lambda i: (_i32(0), _i32(i), _i32(0))),
        ],
        out_shape=[jax.ShapeDtypeStruct((N, H), jnp.float32),
                   jax.ShapeDtypeStruct((_NC, N, Hh), jnp.float32)],
    )(x, W1, degT)


def _tc_stage2(acc1, h1, degT, b1, W2, Dp):
    N, H = h1.shape
    Hh = H // _NC
    C2 = W2.shape[1]
    nb = N // _BR

    def body(a_ref, h_ref, d_ref, b_ref, w_ref, h2_ref, g2_ref):
        s = jnp.concatenate([a_ref[0], a_ref[1]], axis=1)
        d = d_ref[...]
        dinv = lax.rsqrt(d[:, 0:1] + d[:, 1:2] + 1.0)
        z = dinv * s + (dinv * dinv) * h_ref[...] + b_ref[...]
        hh = jnp.maximum(z, 0.0)
        h2 = jnp.dot(hh, w_ref[...], preferred_element_type=jnp.float32)
        h2_ref[...] = h2
        g2 = h2 * dinv
        if Dp > C2:
            g2 = jnp.concatenate(
                [g2, jnp.zeros((_BR, Dp - C2), jnp.float32)], axis=1)
        g2_ref[...] = g2

    return pl.pallas_call(
        body,
        grid=(nb,),
        in_specs=[
            pl.BlockSpec((_NC, _BR, Hh),
                         lambda i: (_i32(0), _i32(i), _i32(0))),
            pl.BlockSpec((_BR, H), lambda i: (_i32(i), _i32(0))),
            pl.BlockSpec((_BR, _NC), lambda i: (_i32(i), _i32(0))),
            pl.BlockSpec((1, H), lambda i: (_i32(0), _i32(0))),
            pl.BlockSpec((H, C2), lambda i: (_i32(0), _i32(0))),
        ],
        out_specs=[
            pl.BlockSpec((_BR, C2), lambda i: (_i32(i), _i32(0))),
            pl.BlockSpec((_BR, Dp), lambda i: (_i32(i), _i32(0))),
        ],
        out_shape=[jax.ShapeDtypeStruct((N, C2), jnp.float32),
                   jax.ShapeDtypeStruct((N, Dp), jnp.float32)],
    )(acc1, h1, degT, b1, W2)


def _tc_stage3(acc2, h2, degT, b2, Dp):
    N, C2 = h2.shape
    nb = N // _BR

    def body(a_ref, h_ref, d_ref, b_ref, o_ref):
        s = (a_ref[0] + a_ref[1])[:, :C2]
        d = d_ref[...]
        dinv = lax.rsqrt(d[:, 0:1] + d[:, 1:2] + 1.0)
        z = dinv * s + (dinv * dinv) * h_ref[...] + b_ref[...]
        m = jnp.max(z, axis=1, keepdims=True)
        e = jnp.exp(z - m)
        lse = jnp.log(jnp.sum(e, axis=1, keepdims=True)) + m
        o_ref[...] = z - lse

    return pl.pallas_call(
        body,
        grid=(nb,),
        in_specs=[
            pl.BlockSpec((_NC, _BR, Dp),
                         lambda i: (_i32(0), _i32(i), _i32(0))),
            pl.BlockSpec((_BR, C2), lambda i: (_i32(i), _i32(0))),
            pl.BlockSpec((_BR, _NC), lambda i: (_i32(i), _i32(0))),
            pl.BlockSpec((1, C2), lambda i: (_i32(0), _i32(0))),
        ],
        out_specs=pl.BlockSpec((_BR, C2), lambda i: (_i32(i), _i32(0))),
        out_shape=jax.ShapeDtypeStruct((N, C2), jnp.float32),
    )(acc2, h2, degT, b2)


def kernel(x, edge_index, W1, b1, W2, b2):
    N, F = x.shape
    E = edge_index.shape[1]
    H = W1.shape[1]
    C2 = W2.shape[1]
    Dp = 48
    assert N % _BR == 0 and N % 8 == 0
    ept = ((E + _NW * _K * _GSZ - 1) // (_NW * _K * _GSZ)) * _K * _GSZ
    Epad = _NW * ept
    Npad = N + _K

    ei = edge_index.astype(jnp.int32)

    def _tiled(nt):
        rpt = E // nt
        dpt = ept * (_NW // nt) - rpt
        dsrc = jnp.zeros((nt, dpt), jnp.int32)
        ddst = jnp.broadcast_to(
            _i32(N) + (jnp.arange(dpt, dtype=jnp.int32) % _i32(_K)),
            (nt, dpt))
        s = jnp.concatenate([ei[0].reshape(nt, rpt), dsrc], axis=1)
        d = jnp.concatenate([ei[1].reshape(nt, rpt), ddst], axis=1)
        return s.reshape(nt, -1, _K), d.reshape(nt, -1, _K)

    src1, dst1 = _tiled(_NS)
    src2, dst2 = _tiled(_NW)

    degp = _make_deg_kernel(N, Npad)(dst2).reshape(_NC, N)
    degT = jnp.transpose(degp)

    h1, g1 = _tc_stage1(x, W1, degT)
    acc1 = _make_row_scatter_kernel(N, Npad, H // _NC, True)(src1, dst1, g1)
    h2, g2p = _tc_stage2(acc1, h1, degT, b1.reshape(1, H), W2, Dp)
    acc2 = _make_row_scatter_kernel(N, Npad, Dp, False)(src2, dst2, g2p)
    return _tc_stage3(acc2, h2, degT, b2.reshape(1, C2), Dp)


# speedup vs baseline: 18.6398x; 1.3319x over previous
import functools

import jax
import jax.numpy as jnp
from jax import lax
from jax.experimental import pallas as pl
from jax.experimental.pallas import tpu as pltpu
from jax.experimental.pallas import tpu_sc as plsc

_NC = 2
_NS = 16
_NW = _NC * _NS

_K = 128
_CPK = 80
_GSZ = 8
_RING = 2 * _GSZ
_NBUF = 3
_BR = 2000


def _sc_mesh():
    return plsc.VectorSubcoreMesh(
        core_axis_name="c", subcore_axis_name="s",
        num_cores=_NC, num_subcores=_NS)


def _i32(v):
    return jnp.int32(v)


def _stripe(si, N):
    base = si * _i32(640)
    nst = jnp.where(base + _i32(640) <= _i32(N),
                    _i32(640 // _CPK), (_i32(N) - base) // _i32(_CPK))
    return base, nst


def _make_deg_kernel(N, Npad):
    @functools.partial(
        pl.kernel,
        out_type=jax.ShapeDtypeStruct((_NC * N,), jnp.float32),
        mesh=_sc_mesh(),
        scratch_types=[
            pltpu.VMEM((_GSZ, _K), jnp.int32),
            pltpu.VMEM((_K,), jnp.float32),
            pltpu.VMEM((_CPK,), jnp.float32),
            pltpu.VMEM_SHARED((Npad,), jnp.float32),
        ],
    )
    def deg_kernel(dst_hbm, out_hbm, didx, ones_v, zbuf, acc):
        ci = lax.axis_index("c")
        si = lax.axis_index("s")
        wid = ci * _i32(_NS) + si
        ng = dst_hbm.shape[1] // _GSZ
        for j in range(_K // 16):
            ones_v[pl.ds(j * 16, 16)] = jnp.ones((16,), jnp.float32)
        for j in range(_CPK // 16):
            zbuf[pl.ds(j * 16, 16)] = jnp.zeros((16,), jnp.float32)
        base, nst = _stripe(si, N)

        def zcopy(t, carry):
            pltpu.sync_copy(zbuf, acc.at[pl.ds(base + t * _i32(_CPK), _CPK)])
            return carry

        lax.fori_loop(_i32(0), nst, zcopy, _i32(0))
        plsc.subcore_barrier()

        def gbody(g, carry):
            off = pl.multiple_of(g * _i32(_GSZ), _GSZ)
            pltpu.sync_copy(dst_hbm.at[wid, pl.ds(off, _GSZ)], didx)

            def body(j, c2):
                pltpu.sync_copy(ones_v, acc.at[didx.at[j]], add=True)
                return c2

            return lax.fori_loop(_i32(0), _i32(_GSZ), body, carry)

        lax.fori_loop(_i32(0), _i32(ng), gbody, _i32(0))
        plsc.subcore_barrier()

        def obody(t, carry):
            rb = base + t * _i32(_CPK)
            pltpu.sync_copy(acc.at[pl.ds(rb, _CPK)], zbuf)
            pltpu.sync_copy(zbuf, out_hbm.at[pl.ds(ci * _i32(N) + rb, _CPK)])
            return carry

        lax.fori_loop(_i32(0), nst, obody, _i32(0))

    return deg_kernel


def _make_row_scatter_kernel(N, Npad, D, col_split):
    @functools.partial(
        pl.kernel,
        out_type=jax.ShapeDtypeStruct((_NC, N, D), jnp.float32),
        mesh=_sc_mesh(),
        compiler_params=pltpu.CompilerParams(use_tc_tiling_on_sc=False),
        scratch_types=[
            pltpu.VMEM((_RING, _K), jnp.int32),
            pltpu.VMEM((_RING, _K), jnp.int32),
            pltpu.VMEM((_NBUF, _K, D), jnp.float32),
            pltpu.VMEM_SHARED((Npad, D), jnp.float32),
            pltpu.SemaphoreType.DMA,
            pltpu.SemaphoreType.DMA,
            pltpu.SemaphoreType.DMA,
            pltpu.SemaphoreType.DMA,
        ],
    )
    def scat_kernel(src_hbm, dst_hbm, g_hbm, out_hbm,
                    sidx, didx, rows, acc, gsemA, gsemB, isem, ssem):
        ci = lax.axis_index("c")
        si = lax.axis_index("s")
        eix = si if col_split else ci * _i32(_NS) + si
        gtab = g_hbm.at[ci] if col_split else g_hbm
        ch = src_hbm.shape[1]
        ng = ch // _GSZ
        zv = jnp.zeros((16,), jnp.float32)

        def zbody(r, carry):
            for j in range(D // 16):
                rows[_i32(0), r, pl.ds(j * 16, 16)] = zv
            return carry

        lax.fori_loop(_i32(0), _i32(_CPK), zbody, _i32(0))
        base, nst = _stripe(si, N)
        stage = rows.at[_i32(0), pl.ds(0, _CPK)]

        def zcopy(t, carry):
            pltpu.sync_copy(stage,
                            acc.at[pl.ds(base + t * _i32(_CPK), _CPK)])
            return carry

        lax.fori_loop(_i32(0), nst, zcopy, _i32(0))
        plsc.subcore_barrier()

        pltpu.sync_copy(src_hbm.at[eix, pl.ds(0, _GSZ)],
                        sidx.at[pl.ds(0, _GSZ)])
        pltpu.sync_copy(dst_hbm.at[eix, pl.ds(0, _GSZ)],
                        didx.at[pl.ds(0, _GSZ)])
        pltpu.sync_copy(gtab.at[sidx.at[_i32(0)]], rows.at[_i32(0)])
        pltpu.async_copy(gtab.at[sidx.at[_i32(1)]], rows.at[_i32(1)], gsemB)

        def gwait(r, b, parity):
            @pl.when(parity == _i32(0))
            def _():
                pltpu.make_async_copy(gtab.at[sidx.at[r]],
                                      rows.at[b], gsemA).wait()

            @pl.when(parity == _i32(1))
            def _():
                pltpu.make_async_copy(gtab.at[sidx.at[r]],
                                      rows.at[b], gsemB).wait()

        def step(c, carry):
            rc = lax.rem(c, _i32(_RING))
            r2 = lax.rem(c + _i32(2), _i32(_RING))
            b0 = lax.rem(c, _i32(_NBUF))
            b1 = lax.rem(c + _i32(1), _i32(_NBUF))
            b2 = lax.rem(c + _i32(2), _i32(_NBUF))
            p2 = lax.rem(c + _i32(2), _i32(2))

            @pl.when(c > _i32(0))
            def _():
                pltpu.make_async_copy(
                    rows.at[b0], acc.at[didx.at[rc]], ssem).wait()

            @pl.when(p2 == _i32(0))
            def _():
                pltpu.async_copy(gtab.at[sidx.at[r2]], rows.at[b2], gsemA)

            @pl.when(p2 == _i32(1))
            def _():
                pltpu.async_copy(gtab.at[sidx.at[r2]], rows.at[b2], gsemB)

            pltpu.async_copy(rows.at[b0], acc.at[didx.at[rc]], ssem,
                             add=True)
            gwait(lax.rem(c + _i32(1), _i32(_RING)), b1,
                  lax.rem(c + _i32(1), _i32(2)))
            return carry

        def gbody(g, carry):
            gb = lax.rem(g, _i32(2))
            nxt = jnp.where(g + _i32(1) < _i32(ng), g + _i32(1), _i32(0))
            noff = pl.multiple_of(nxt * _i32(_GSZ), _GSZ)
            rdst = (_i32(1) - gb) * _i32(_GSZ)
            d1 = pltpu.async_copy(src_hbm.at[eix, pl.ds(noff, _GSZ)],
                                  sidx.at[pl.ds(rdst, _GSZ)], isem)
            d2 = pltpu.async_copy(dst_hbm.at[eix, pl.ds(noff, _GSZ)],
                                  didx.at[pl.ds(rdst, _GSZ)], isem)
            g8 = g * _i32(_GSZ)

            def ibody(j, c2):
                return step(g8 + j, c2)

            carry = lax.fori_loop(_i32(0), _i32(_GSZ - 2), ibody, carry)
            d1.wait()
            d2.wait()
            carry = step(g8 + _i32(_GSZ - 2), carry)
            return step(g8 + _i32(_GSZ - 1), carry)

        lax.fori_loop(_i32(0), _i32(ng), gbody, _i32(0))
        pltpu.make_async_copy(
            rows.at[_i32(0)], acc.at[didx.at[_i32(0)]], ssem).wait()
        gwait(_i32(0), _i32(1), lax.rem(_i32(ng * _GSZ + 1), _i32(2)))
        plsc.subcore_barrier()

        def obody(t, carry):
            rb = base + t * _i32(_CPK)
            pltpu.sync_copy(acc.at[pl.ds(rb, _CPK)], stage)
            pltpu.sync_copy(stage, out_hbm.at[ci, pl.ds(rb, _CPK)])
            return carry

        lax.fori_loop(_i32(0), nst, obody, _i32(0))

    return scat_kernel


def _tc_stage1(x, W1, degT):
    N, F = x.shape
    H = W1.shape[1]
    Hh = H // _NC
    nb = N // _BR

    def body(x_ref, w_ref, d_ref, h_ref, g_ref):
        h = jnp.dot(x_ref[...], w_ref[...], preferred_element_type=jnp.float32)
        d = d_ref[...]
        dinv = lax.rsqrt(d[:, 0:1] + d[:, 1:2] + 1.0)
        h_ref[...] = h
        g = h * dinv
        g_ref[0] = g[:, :Hh]
        g_ref[1] = g[:, Hh:]

    return pl.pallas_call(
        body,
        grid=(nb,),
        in_specs=[
            pl.BlockSpec((_BR, F), lambda i: (_i32(i), _i32(0))),
            pl.BlockSpec((F, H), lambda i: (_i32(0), _i32(0))),
            pl.BlockSpec((_BR, _NC), lambda i: (_i32(i), _i32(0))),
        ],
        out_specs=[
            pl.BlockSpec((_BR, H), lambda i: (_i32(i), _i32(0))),
            pl.BlockSpec((_NC, _BR, Hh),
                         lambda i: (_i32(0), _i32(i), _i32(0))),
        ],
        out_shape=[jax.ShapeDtypeStruct((N, H), jnp.float32),
                   jax.ShapeDtypeStruct((_NC, N, Hh), jnp.float32)],
    )(x, W1, degT)


def _tc_stage2(acc1, h1, degT, b1, W2, Dp):
    N, H = h1.shape
    Hh = H // _NC
    C2 = W2.shape[1]
    nb = N // _BR

    def body(a_ref, h_ref, d_ref, b_ref, w_ref, h2_ref, g2_ref):
        s = jnp.concatenate([a_ref[0], a_ref[1]], axis=1)
        d = d_ref[...]
        dinv = lax.rsqrt(d[:, 0:1] + d[:, 1:2] + 1.0)
        z = dinv * s + (dinv * dinv) * h_ref[...] + b_ref[...]
        hh = jnp.maximum(z, 0.0)
        h2 = jnp.dot(hh, w_ref[...], preferred_element_type=jnp.float32)
        h2_ref[...] = h2
        g2 = h2 * dinv
        if Dp > C2:
            g2 = jnp.concatenate(
                [g2, jnp.zeros((_BR, Dp - C2), jnp.float32)], axis=1)
        g2_ref[...] = g2

    return pl.pallas_call(
        body,
        grid=(nb,),
        in_specs=[
            pl.BlockSpec((_NC, _BR, Hh),
                         lambda i: (_i32(0), _i32(i), _i32(0))),
            pl.BlockSpec((_BR, H), lambda i: (_i32(i), _i32(0))),
            pl.BlockSpec((_BR, _NC), lambda i: (_i32(i), _i32(0))),
            pl.BlockSpec((1, H), lambda i: (_i32(0), _i32(0))),
            pl.BlockSpec((H, C2), lambda i: (_i32(0), _i32(0))),
        ],
        out_specs=[
            pl.BlockSpec((_BR, C2), lambda i: (_i32(i), _i32(0))),
            pl.BlockSpec((_BR, Dp), lambda i: (_i32(i), _i32(0))),
        ],
        out_shape=[jax.ShapeDtypeStruct((N, C2), jnp.float32),
                   jax.ShapeDtypeStruct((N, Dp), jnp.float32)],
    )(acc1, h1, degT, b1, W2)


def _tc_stage3(acc2, h2, degT, b2, Dp):
    N, C2 = h2.shape
    nb = N // _BR

    def body(a_ref, h_ref, d_ref, b_ref, o_ref):
        s = (a_ref[0] + a_ref[1])[:, :C2]
        d = d_ref[...]
        dinv = lax.rsqrt(d[:, 0:1] + d[:, 1:2] + 1.0)
        z = dinv * s + (dinv * dinv) * h_ref[...] + b_ref[...]
        m = jnp.max(z, axis=1, keepdims=True)
        e = jnp.exp(z - m)
        lse = jnp.log(jnp.sum(e, axis=1, keepdims=True)) + m
        o_ref[...] = z - lse

    return pl.pallas_call(
        body,
        grid=(nb,),
        in_specs=[
            pl.BlockSpec((_NC, _BR, Dp),
                         lambda i: (_i32(0), _i32(i), _i32(0))),
            pl.BlockSpec((_BR, C2), lambda i: (_i32(i), _i32(0))),
            pl.BlockSpec((_BR, _NC), lambda i: (_i32(i), _i32(0))),
            pl.BlockSpec((1, C2), lambda i: (_i32(0), _i32(0))),
        ],
        out_specs=pl.BlockSpec((_BR, C2), lambda i: (_i32(i), _i32(0))),
        out_shape=jax.ShapeDtypeStruct((N, C2), jnp.float32),
    )(acc2, h2, degT, b2)


def kernel(x, edge_index, W1, b1, W2, b2):
    N, F = x.shape
    E = edge_index.shape[1]
    H = W1.shape[1]
    C2 = W2.shape[1]
    Dp = 48
    assert N % _BR == 0 and N % 8 == 0
    ept = ((E + _NW * _K * _GSZ - 1) // (_NW * _K * _GSZ)) * _K * _GSZ
    Epad = _NW * ept
    Npad = N + _K

    ei = edge_index.astype(jnp.int32)

    def _tiled(nt):
        rpt = E // nt
        dpt = ept * (_NW // nt) - rpt
        dsrc = jnp.zeros((nt, dpt), jnp.int32)
        ddst = jnp.broadcast_to(
            _i32(N) + (jnp.arange(dpt, dtype=jnp.int32) % _i32(_K)),
            (nt, dpt))
        s = jnp.concatenate([ei[0].reshape(nt, rpt), dsrc], axis=1)
        d = jnp.concatenate([ei[1].reshape(nt, rpt), ddst], axis=1)
        return s.reshape(nt, -1, _K), d.reshape(nt, -1, _K)

    src1, dst1 = _tiled(_NS)
    src2, dst2 = _tiled(_NW)

    degp = _make_deg_kernel(N, Npad)(dst2).reshape(_NC, N)
    degT = jnp.transpose(degp)

    h1, g1 = _tc_stage1(x, W1, degT)
    acc1 = _make_row_scatter_kernel(N, Npad, H // _NC, True)(src1, dst1, g1)
    h2, g2p = _tc_stage2(acc1, h1, degT, b1.reshape(1, H), W2, Dp)
    acc2 = _make_row_scatter_kernel(N, Npad, Dp, False)(src2, dst2, g2p)
    return _tc_stage3(acc2, h2, degT, b2.reshape(1, C2), Dp)


# speedup vs baseline: 18.6569x; 1.0009x over previous
import functools

import jax
import jax.numpy as jnp
from jax import lax
from jax.experimental import pallas as pl
from jax.experimental.pallas import tpu as pltpu
from jax.experimental.pallas import tpu_sc as plsc

_NC = 2
_NS = 16
_NW = _NC * _NS

_K = 128
_CPK = 80
_GSZ = 4
_RING = 2 * _GSZ
_NPR = 48
_BR = 2000


def _sc_mesh():
    return plsc.VectorSubcoreMesh(
        core_axis_name="c", subcore_axis_name="s",
        num_cores=_NC, num_subcores=_NS)


def _i32(v):
    return jnp.int32(v)


def _stripe(si, N):
    base = si * _i32(640)
    nst = jnp.where(base + _i32(640) <= _i32(N),
                    _i32(640 // _CPK), (_i32(N) - base) // _i32(_CPK))
    return base, nst


def _make_deg_kernel(N, Npad):
    @functools.partial(
        pl.kernel,
        out_type=jax.ShapeDtypeStruct((_NC * N,), jnp.float32),
        mesh=_sc_mesh(),
        compiler_params=pltpu.CompilerParams(use_tc_tiling_on_sc=False),
        scratch_types=[
            pltpu.VMEM((_GSZ, _K), jnp.int32),
            pltpu.VMEM((_K,), jnp.float32),
            pltpu.VMEM((_CPK,), jnp.float32),
            pltpu.VMEM_SHARED((Npad,), jnp.float32),
        ],
    )
    def deg_kernel(dst_hbm, out_hbm, didx, ones_v, zbuf, acc):
        ci = lax.axis_index("c")
        si = lax.axis_index("s")
        wid = ci * _i32(_NS) + si
        ng = dst_hbm.shape[1] // _GSZ
        for j in range(_K // 16):
            ones_v[pl.ds(j * 16, 16)] = jnp.ones((16,), jnp.float32)
        for j in range(_CPK // 16):
            zbuf[pl.ds(j * 16, 16)] = jnp.zeros((16,), jnp.float32)
        base, nst = _stripe(si, N)

        def zcopy(t, carry):
            pltpu.sync_copy(zbuf, acc.at[pl.ds(base + t * _i32(_CPK), _CPK)])
            return carry

        lax.fori_loop(_i32(0), nst, zcopy, _i32(0))
        plsc.subcore_barrier()

        def gbody(g, carry):
            off = pl.multiple_of(g * _i32(_GSZ), _GSZ)
            pltpu.sync_copy(dst_hbm.at[wid, pl.ds(off, _GSZ)], didx)

            def body(j, c2):
                pltpu.sync_copy(ones_v, acc.at[didx.at[j]], add=True)
                return c2

            return lax.fori_loop(_i32(0), _i32(_GSZ), body, carry)

        lax.fori_loop(_i32(0), _i32(ng), gbody, _i32(0))
        plsc.subcore_barrier()

        def obody(t, carry):
            rb = base + t * _i32(_CPK)
            pltpu.sync_copy(acc.at[pl.ds(rb, _CPK)], zbuf)
            pltpu.sync_copy(zbuf, out_hbm.at[pl.ds(ci * _i32(N) + rb, _CPK)])
            return carry

        lax.fori_loop(_i32(0), nst, obody, _i32(0))

    return deg_kernel


def _make_row_scatter_kernel(N, Npad, D, col_split, gd):
    nbuf = gd + 1

    @functools.partial(
        pl.kernel,
        out_type=jax.ShapeDtypeStruct((_NC, N, D), jnp.float32),
        mesh=_sc_mesh(),
        compiler_params=pltpu.CompilerParams(use_tc_tiling_on_sc=False),
        scratch_types=[
            pltpu.VMEM((_RING, _K), jnp.int32),
            pltpu.VMEM((_RING, _K), jnp.int32),
            pltpu.VMEM((nbuf, _K, D), jnp.float32),
            pltpu.VMEM_SHARED((Npad, D), jnp.float32),
            [pltpu.SemaphoreType.DMA] * gd,
            pltpu.SemaphoreType.DMA,
            pltpu.SemaphoreType.DMA,
        ],
    )
    def scat_kernel(src_hbm, dst_hbm, g_hbm, out_hbm,
                    sidx, didx, rows, acc, gsems, isem, ssem):
        ci = lax.axis_index("c")
        si = lax.axis_index("s")
        eix = si if col_split else ci * _i32(_NS) + si
        gtab = g_hbm.at[ci] if col_split else g_hbm
        ch = src_hbm.shape[1]
        ng = ch // _GSZ
        zv = jnp.zeros((16,), jnp.float32)

        def zbody(r, carry):
            for j in range(D // 16):
                rows[_i32(0), r, pl.ds(j * 16, 16)] = zv
            return carry

        lax.fori_loop(_i32(0), _i32(_CPK), zbody, _i32(0))
        base, nst = _stripe(si, N)
        stage = rows.at[_i32(0), pl.ds(0, _CPK)]

        def zcopy(t, carry):
            pltpu.sync_copy(stage,
                            acc.at[pl.ds(base + t * _i32(_CPK), _CPK)])
            return carry

        lax.fori_loop(_i32(0), nst, zcopy, _i32(0))
        plsc.subcore_barrier()

        def gissue(r, b, sem):
            pltpu.async_copy(gtab.at[sidx.at[r]], rows.at[b], sem)

        def gwait(sem):
            pltpu.make_async_copy(gtab.at[sidx.at[_i32(0)]],
                                  rows.at[_i32(0)], sem).wait()

        pltpu.sync_copy(src_hbm.at[eix, pl.ds(0, _GSZ)],
                        sidx.at[pl.ds(0, _GSZ)])
        pltpu.sync_copy(dst_hbm.at[eix, pl.ds(0, _GSZ)],
                        didx.at[pl.ds(0, _GSZ)])
        pltpu.sync_copy(gtab.at[sidx.at[_i32(0)]], rows.at[_i32(0)])
        for k in range(1, gd):
            gissue(_i32(k), _i32(k), gsems[k % gd])

        def step(c, carry):
            rc = lax.rem(c, _i32(_RING))
            rg = lax.rem(c + _i32(gd), _i32(_RING))
            b0 = lax.rem(c, _i32(nbuf))
            bg = lax.rem(c + _i32(gd), _i32(nbuf))
            pg = lax.rem(c + _i32(gd), _i32(gd))

            @pl.when(c > _i32(0))
            def _():
                pltpu.make_async_copy(
                    rows.at[b0], acc.at[didx.at[rc]], ssem).wait()

            for k in range(gd):
                @pl.when(pg == _i32(k))
                def _(k=k):
                    gissue(rg, bg, gsems[k])

            pltpu.async_copy(rows.at[b0], acc.at[didx.at[rc]], ssem,
                             add=True)
            p1 = lax.rem(c + _i32(1), _i32(gd))
            for k in range(gd):
                @pl.when(p1 == _i32(k))
                def _(k=k):
                    gwait(gsems[k])

            return carry

        def gbody(g, carry):
            gb = lax.rem(g, _i32(2))
            nxt = jnp.where(g + _i32(1) < _i32(ng), g + _i32(1), _i32(0))
            noff = pl.multiple_of(nxt * _i32(_GSZ), _GSZ)
            rdst = (_i32(1) - gb) * _i32(_GSZ)
            d1 = pltpu.async_copy(src_hbm.at[eix, pl.ds(noff, _GSZ)],
                                  sidx.at[pl.ds(rdst, _GSZ)], isem)
            d2 = pltpu.async_copy(dst_hbm.at[eix, pl.ds(noff, _GSZ)],
                                  didx.at[pl.ds(rdst, _GSZ)], isem)
            g8 = g * _i32(_GSZ)

            def ibody(j, c2):
                return step(g8 + j, c2)

            carry = lax.fori_loop(_i32(0), _i32(_GSZ - gd), ibody, carry)
            d1.wait()
            d2.wait()
            for j in range(_GSZ - gd, _GSZ):
                carry = step(g8 + _i32(j), carry)
            return carry

        lax.fori_loop(_i32(0), _i32(ng), gbody, _i32(0))
        pltpu.make_async_copy(
            rows.at[_i32(0)], acc.at[didx.at[_i32(0)]], ssem).wait()
        last = ng * _GSZ
        for t in range(1, gd):
            gwait(gsems[(last + t) % gd])
        plsc.subcore_barrier()

        def obody(t, carry):
            rb = base + t * _i32(_CPK)
            pltpu.sync_copy(acc.at[pl.ds(rb, _CPK)], stage)
            pltpu.sync_copy(stage, out_hbm.at[ci, pl.ds(rb, _CPK)])
            return carry

        lax.fori_loop(_i32(0), nst, obody, _i32(0))

    return scat_kernel


def _tc_stage1(x, W1, degT):
    N, F = x.shape
    H = W1.shape[1]
    Hh = H // _NC
    nb = N // _BR

    def body(x_ref, w_ref, d_ref, h_ref, g_ref):
        h = jnp.dot(x_ref[...], w_ref[...], preferred_element_type=jnp.float32)
        d = d_ref[...]
        dinv = lax.rsqrt(d[:, 0:1] + d[:, 1:2] + 1.0)
        h_ref[...] = h
        g = h * dinv
        g_ref[0] = g[:, :Hh]
        g_ref[1] = g[:, Hh:]

    return pl.pallas_call(
        body,
        grid=(nb,),
        in_specs=[
            pl.BlockSpec((_BR, F), lambda i: (_i32(i), _i32(0))),
            pl.BlockSpec((F, H), lambda i: (_i32(0), _i32(0))),
            pl.BlockSpec((_BR, _NC), lambda i: (_i32(i), _i32(0))),
        ],
        out_specs=[
            pl.BlockSpec((_BR, H), lambda i: (_i32(i), _i32(0))),
            pl.BlockSpec((_NC, _BR, Hh),
                         lambda i: (_i32(0), _i32(i), _i32(0))),
        ],
        out_shape=[jax.ShapeDtypeStruct((N, H), jnp.float32),
                   jax.ShapeDtypeStruct((_NC, N, Hh), jnp.float32)],
    )(x, W1, degT)


def _tc_stage2(acc1, h1, degT, b1, W2, Dp):
    N, H = h1.shape
    Hh = H // _NC
    C2 = W2.shape[1]
    nb = N // _BR

    def body(a_ref, h_ref, d_ref, b_ref, w_ref, h2_ref, g2_ref):
        s = jnp.concatenate([a_ref[0], a_ref[1]], axis=1)
        d = d_ref[...]
        dinv = lax.rsqrt(d[:, 0:1] + d[:, 1:2] + 1.0)
        z = dinv * s + (dinv * dinv) * h_ref[...] + b_ref[...]
        hh = jnp.maximum(z, 0.0)
        h2 = jnp.dot(hh, w_ref[...], preferred_element_type=jnp.float32)
        h2_ref[...] = h2
        g2 = h2 * dinv
        if Dp > C2:
            g2 = jnp.concatenate(
                [g2, jnp.zeros((_BR, Dp - C2), jnp.float32)], axis=1)
        g2_ref[...] = g2

    return pl.pallas_call(
        body,
        grid=(nb,),
        in_specs=[
            pl.BlockSpec((_NC, _BR, Hh),
                         lambda i: (_i32(0), _i32(i), _i32(0))),
            pl.BlockSpec((_BR, H), lambda i: (_i32(i), _i32(0))),
            pl.BlockSpec((_BR, _NC), lambda i: (_i32(i), _i32(0))),
            pl.BlockSpec((1, H), lambda i: (_i32(0), _i32(0))),
            pl.BlockSpec((H, C2), lambda i: (_i32(0), _i32(0))),
        ],
        out_specs=[
            pl.BlockSpec((_BR, C2), lambda i: (_i32(i), _i32(0))),
            pl.BlockSpec((_BR, Dp), lambda i: (_i32(i), _i32(0))),
        ],
        out_shape=[jax.ShapeDtypeStruct((N, C2), jnp.float32),
                   jax.ShapeDtypeStruct((N, Dp), jnp.float32)],
    )(acc1, h1, degT, b1, W2)


def _tc_stage3(acc2, h2, degT, b2, Dp):
    N, C2 = h2.shape
    nb = N // _BR

    def body(a_ref, h_ref, d_ref, b_ref, o_ref):
        s = (a_ref[0] + a_ref[1])[:, :C2]
        d = d_ref[...]
        dinv = lax.rsqrt(d[:, 0:1] + d[:, 1:2] + 1.0)
        z = dinv * s + (dinv * dinv) * h_ref[...] + b_ref[...]
        m = jnp.max(z, axis=1, keepdims=True)
        e = jnp.exp(z - m)
        lse = jnp.log(jnp.sum(e, axis=1, keepdims=True)) + m
        o_ref[...] = z - lse

    return pl.pallas_call(
        body,
        grid=(nb,),
        in_specs=[
            pl.BlockSpec((_NC, _BR, Dp),
                         lambda i: (_i32(0), _i32(i), _i32(0))),
            pl.BlockSpec((_BR, C2), lambda i: (_i32(i), _i32(0))),
            pl.BlockSpec((_BR, _NC), lambda i: (_i32(i), _i32(0))),
            pl.BlockSpec((1, C2), lambda i: (_i32(0), _i32(0))),
        ],
        out_specs=pl.BlockSpec((_BR, C2), lambda i: (_i32(i), _i32(0))),
        out_shape=jax.ShapeDtypeStruct((N, C2), jnp.float32),
    )(acc2, h2, degT, b2)


def kernel(x, edge_index, W1, b1, W2, b2):
    N, F = x.shape
    E = edge_index.shape[1]
    H = W1.shape[1]
    C2 = W2.shape[1]
    Dp = 48
    assert N % _BR == 0 and N % 8 == 0
    ept = ((E + _NW * _K * _GSZ - 1) // (_NW * _K * _GSZ)) * _K * _GSZ
    Epad = _NW * ept
    Npad = N + _NPR

    ei = edge_index.astype(jnp.int32)

    def _tiled(nt):
        rpt = E // nt
        dpt = ept * (_NW // nt) - rpt
        dsrc = jnp.zeros((nt, dpt), jnp.int32)
        ddst = jnp.broadcast_to(
            _i32(N) + (jnp.arange(dpt, dtype=jnp.int32) % _i32(_NPR)),
            (nt, dpt))
        s = jnp.concatenate([ei[0].reshape(nt, rpt), dsrc], axis=1)
        d = jnp.concatenate([ei[1].reshape(nt, rpt), ddst], axis=1)
        return s.reshape(nt, -1, _K), d.reshape(nt, -1, _K)

    src1, dst1 = _tiled(_NS)
    src2, dst2 = _tiled(_NW)

    degp = _make_deg_kernel(N, Npad)(dst2).reshape(_NC, N)
    degT = jnp.transpose(degp)

    h1, g1 = _tc_stage1(x, W1, degT)
    acc1 = _make_row_scatter_kernel(N, Npad, H // _NC, True, 3)(src1, dst1, g1)
    h2, g2p = _tc_stage2(acc1, h1, degT, b1.reshape(1, H), W2, Dp)
    acc2 = _make_row_scatter_kernel(N, Npad, Dp, False, 2)(src2, dst2, g2p)
    return _tc_stage3(acc2, h2, degT, b2.reshape(1, C2), Dp)
